# baseline (speedup 1.0000x reference)
import functools

import jax
import jax.numpy as jnp
from jax import lax
from jax.experimental import pallas as pl
from jax.experimental.pallas import tpu as pltpu
from jax.experimental.pallas import tpu_sc as plsc


def _sc_body(col_hbm, row_hbm, out_hbm, colblk, rowblk, sems):
    wid = lax.axis_index("s") * 2 + lax.axis_index("c")
    b = wid // 4
    q = wid % 4
    pltpu.sync_copy(col_hbm.at[pl.ds(0, 32)], colblk)
    pltpu.sync_copy(row_hbm.at[pl.ds(0, 32)], rowblk)
    copies = []
    for j in range(8):
        hw = q * 8 + j
        copies.append(
            pltpu.make_async_copy(
                colblk, out_hbm.at[b, hw, :, pl.ds(0, 256)], sems.at[0, j]
            )
        )
        copies.append(
            pltpu.make_async_copy(
                rowblk, out_hbm.at[b, :, hw, pl.ds(256, 256)], sems.at[1, j]
            )
        )
    for c in copies:
        c.start()
    for c in copies:
        c.wait()


def kernel(x, mask, row_embed, col_embed):
    b = x.shape[0]
    h, w = x.shape[-2], x.shape[-1]
    d = col_embed.shape[-1]
    sc_call = functools.partial(
        pl.kernel,
        out_type=jax.ShapeDtypeStruct((b, h, w, 2 * d), jnp.float32),
        mesh=plsc.VectorSubcoreMesh(core_axis_name="c", subcore_axis_name="s"),
        scratch_types=[
            pltpu.VMEM((w, d), jnp.float32),
            pltpu.VMEM((h, d), jnp.float32),
            pltpu.SemaphoreType.DMA((2, 8)),
        ],
    )(_sc_body)
    out_nat = sc_call(col_embed, row_embed)
    return jnp.transpose(out_nat, (0, 3, 1, 2))


# speedup vs baseline: 4.9059x; 4.9059x over previous
import jax
import jax.numpy as jnp
from jax.experimental import pallas as pl
from jax.experimental.pallas import tpu as pltpu


def _make_body(b):
    def _body(col_ref, row_ref, o_ref, scratch, sems):
        w, d = col_ref.shape
        h = row_ref.shape[0]
        r = scratch.shape[0]
        scratch[:, :, :, :d] = jnp.broadcast_to(
            col_ref[...][None, None, :, :], (r, h, w, d)
        )
        scratch[:, :, :, d:] = jnp.broadcast_to(
            row_ref[...][None, :, None, :], (r, h, w, d)
        )
        copies = [
            pltpu.make_async_copy(scratch, o_ref.at[pl.ds(i * r, r)], sems.at[i])
            for i in range(b // r)
        ]
        for c in copies:
            c.start()
        for c in copies:
            c.wait()

    return _body


def kernel(x, mask, row_embed, col_embed):
    b = x.shape[0]
    h, w = x.shape[-2], x.shape[-1]
    d = col_embed.shape[-1]
    r = 2
    out_nat = pl.pallas_call(
        _make_body(b),
        grid=(1,),
        in_specs=[
            pl.BlockSpec((w, d), lambda i: (0, 0)),
            pl.BlockSpec((h, d), lambda i: (0, 0)),
        ],
        out_specs=pl.BlockSpec(memory_space=pl.ANY),
        out_shape=jax.ShapeDtypeStruct((b, h, w, 2 * d), jnp.float32),
        scratch_shapes=[
            pltpu.VMEM((r, h, w, 2 * d), jnp.float32),
            pltpu.SemaphoreType.DMA((b // r,)),
        ],
    )(col_embed, row_embed)
    return jnp.transpose(out_nat, (0, 3, 1, 2))


# speedup vs baseline: 5.0723x; 1.0339x over previous
import jax
import jax.numpy as jnp
from jax.experimental import pallas as pl
from jax.experimental.pallas import tpu as pltpu

_NQ = 4


def _make_body(b):
    def _body(col_ref, row_ref, o_ref, scratch, sems):
        w, d = col_ref.shape
        h = row_ref.shape[0]
        hq = h // _NQ
        col = col_ref[...]
        row = row_ref[...]
        copies = []
        for q in range(_NQ):
            hs = pl.ds(q * hq, hq)
            scratch[hs, :, :d] = jnp.broadcast_to(col[None, :, :], (hq, w, d))
            scratch[hs, :, d:] = jnp.broadcast_to(
                row[q * hq : (q + 1) * hq, None, :], (hq, w, d)
            )
            for i in range(b):
                c = pltpu.make_async_copy(
                    scratch.at[hs], o_ref.at[i, hs], sems.at[q, i]
                )
                c.start()
                copies.append(c)
        for c in copies:
            c.wait()

    return _body


def kernel(x, mask, row_embed, col_embed):
    b = x.shape[0]
    h, w = x.shape[-2], x.shape[-1]
    d = col_embed.shape[-1]
    out_nat = pl.pallas_call(
        _make_body(b),
        grid=(1,),
        in_specs=[
            pl.BlockSpec((w, d), lambda i: (0, 0)),
            pl.BlockSpec((h, d), lambda i: (0, 0)),
        ],
        out_specs=pl.BlockSpec(memory_space=pl.ANY),
        out_shape=jax.ShapeDtypeStruct((b, h, w, 2 * d), jnp.float32),
        scratch_shapes=[
            pltpu.VMEM((h, w, 2 * d), jnp.float32),
            pltpu.SemaphoreType.DMA((_NQ, b)),
        ],
    )(col_embed, row_embed)
    return jnp.transpose(out_nat, (0, 3, 1, 2))


# speedup vs baseline: 5.0977x; 1.0050x over previous
import jax
import jax.numpy as jnp
from jax.experimental import pallas as pl
from jax.experimental.pallas import tpu as pltpu


def _make_body(b, h, w, d):
    def _body(col_hbm, row_hbm, o_ref, colv, rowv, scratch, insems, sems):
        fetch_col = pltpu.make_async_copy(
            col_hbm.at[pl.ds(0, w)], colv, insems.at[0]
        )
        fetch_row = pltpu.make_async_copy(
            row_hbm.at[pl.ds(0, h)], rowv, insems.at[1]
        )
        fetch_col.start()
        fetch_row.start()
        fetch_col.wait()
        scratch[:, :, :d] = jnp.broadcast_to(colv[...][None, :, :], (h, w, d))
        fetch_row.wait()
        scratch[:, :, d:] = jnp.broadcast_to(rowv[...][:, None, :], (h, w, d))
        copies = [
            pltpu.make_async_copy(scratch, o_ref.at[i], sems.at[i])
            for i in range(b)
        ]
        for c in copies:
            c.start()
        for c in copies:
            c.wait()

    return _body


def kernel(x, mask, row_embed, col_embed):
    b = x.shape[0]
    h, w = x.shape[-2], x.shape[-1]
    d = col_embed.shape[-1]
    out_nat = pl.pallas_call(
        _make_body(b, h, w, d),
        grid=(1,),
        in_specs=[
            pl.BlockSpec(memory_space=pl.ANY),
            pl.BlockSpec(memory_space=pl.ANY),
        ],
        out_specs=pl.BlockSpec(memory_space=pl.ANY),
        out_shape=jax.ShapeDtypeStruct((b, h, w, 2 * d), jnp.float32),
        scratch_shapes=[
            pltpu.VMEM((w, d), jnp.float32),
            pltpu.VMEM((h, d), jnp.float32),
            pltpu.VMEM((h, w, 2 * d), jnp.float32),
            pltpu.SemaphoreType.DMA((2,)),
            pltpu.SemaphoreType.DMA((b,)),
        ],
    )(col_embed, row_embed)
    return jnp.transpose(out_nat, (0, 3, 1, 2))
